# baseline (speedup 1.0000x reference)
import functools

import jax
import jax.numpy as jnp
from jax import lax
from jax.experimental import pallas as pl
from jax.experimental.pallas import tpu as pltpu
from jax.experimental.pallas import tpu_sc as plsc

VOCAB = 100000
CTX = 20
NDIM = 64
HID = 128
BATCH = 1024

NC = 2
NS = 16
NW = NC * NS
N_IDX = BATCH * CTX
CHUNK = 128
CHUNKS_PER_W = N_IDX // (NW * CHUNK)
ROWS_PER_W = CHUNKS_PER_W * CHUNK

TILE_V = 2048
GRID_V = (VOCAB + TILE_V - 1) // TILE_V


def _gather_kernel(x_hbm, table_hbm, out_hbm, idx_v, rows_v, sem):
    wid = lax.axis_index("s") * NC + lax.axis_index("c")
    base = wid * CHUNKS_PER_W
    pltpu.sync_copy(x_hbm.at[wid], idx_v)
    copies = [
        pltpu.async_copy(table_hbm.at[idx_v.at[j]], rows_v.at[j], sem)
        for j in range(CHUNKS_PER_W)
    ]
    for c in copies:
        c.wait()
    pltpu.sync_copy(rows_v, out_hbm.at[pl.ds(base, CHUNKS_PER_W)])


def _sc_gather(x_flat, emb_table):
    mesh = plsc.VectorSubcoreMesh(core_axis_name="c", subcore_axis_name="s")
    k = functools.partial(
        pl.kernel,
        mesh=mesh,
        out_type=jax.ShapeDtypeStruct((NW * CHUNKS_PER_W, CHUNK, NDIM),
                                      jnp.float32),
        scratch_types=[
            pltpu.VMEM((CHUNKS_PER_W, CHUNK), jnp.int32),
            pltpu.VMEM((CHUNKS_PER_W, CHUNK, NDIM), jnp.float32),
            pltpu.SemaphoreType.DMA,
        ],
        compiler_params=pltpu.CompilerParams(use_tc_tiling_on_sc=False),
    )(_gather_kernel)
    return k(x_flat.reshape(NW, CHUNKS_PER_W, CHUNK), emb_table)


def _mlp_kernel(emb_ref, w1_ref, b1_ref, w2_ref, b2_ref, out_ref, h_ref):
    @pl.when(pl.program_id(0) == 0)
    def _():
        h = jnp.dot(emb_ref[...], w1_ref[...],
                    preferred_element_type=jnp.float32)
        h_ref[...] = jnp.maximum(h + b1_ref[...], 0.0)

    out_ref[...] = (
        jnp.dot(h_ref[...], w2_ref[...], preferred_element_type=jnp.float32)
        + b2_ref[...]
    )


def _tc_mlp(emb, W1, b1, W2, b2):
    return pl.pallas_call(
        _mlp_kernel,
        grid=(GRID_V,),
        in_specs=[
            pl.BlockSpec((BATCH, CTX * NDIM), lambda i: (0, 0)),
            pl.BlockSpec((CTX * NDIM, HID), lambda i: (0, 0)),
            pl.BlockSpec((1, HID), lambda i: (0, 0)),
            pl.BlockSpec((HID, TILE_V), lambda i: (0, i)),
            pl.BlockSpec((1, TILE_V), lambda i: (0, i)),
        ],
        out_specs=pl.BlockSpec((BATCH, TILE_V), lambda i: (0, i)),
        out_shape=jax.ShapeDtypeStruct((BATCH, VOCAB), jnp.float32),
        scratch_shapes=[pltpu.VMEM((BATCH, HID), jnp.float32)],
    )(emb, W1, b1.reshape(1, HID), W2, b2.reshape(1, VOCAB))


def kernel(x, emb_table, W1, b1, W2, b2):
    rows = _sc_gather(x.reshape(-1), emb_table)
    emb = rows.reshape(BATCH, CTX * NDIM)
    return _tc_mlp(emb, W1, b1, W2, b2)


# speedup vs baseline: 1.0004x; 1.0004x over previous
import functools

import jax
import jax.numpy as jnp
from jax import lax
from jax.experimental import pallas as pl
from jax.experimental.pallas import tpu as pltpu
from jax.experimental.pallas import tpu_sc as plsc

VOCAB = 100000
CTX = 20
NDIM = 64
HID = 128
BATCH = 1024

NC = 2
NS = 16
NW = NC * NS
N_IDX = BATCH * CTX
CHUNK = 128
CHUNKS_PER_W = N_IDX // (NW * CHUNK)
ROWS_PER_W = CHUNKS_PER_W * CHUNK

TILE_V = 2048
GRID_V = (VOCAB + TILE_V - 1) // TILE_V


def _gather_kernel(x_hbm, table_hbm, out_hbm, idx_v, rows_v, sem):
    wid = lax.axis_index("s") * NC + lax.axis_index("c")
    base = wid * CHUNKS_PER_W
    pltpu.sync_copy(x_hbm.at[wid], idx_v)
    copies = [
        pltpu.async_copy(table_hbm.at[idx_v.at[j]], rows_v.at[j], sem)
        for j in range(CHUNKS_PER_W)
    ]
    for c in copies:
        c.wait()
    pltpu.sync_copy(rows_v, out_hbm.at[pl.ds(base, CHUNKS_PER_W)])


def _sc_gather(x_flat, emb_table):
    mesh = plsc.VectorSubcoreMesh(core_axis_name="c", subcore_axis_name="s")
    k = functools.partial(
        pl.kernel,
        mesh=mesh,
        out_type=jax.ShapeDtypeStruct((NW * CHUNKS_PER_W, CHUNK, NDIM),
                                      jnp.float32),
        scratch_types=[
            pltpu.VMEM((CHUNKS_PER_W, CHUNK), jnp.int32),
            pltpu.VMEM((CHUNKS_PER_W, CHUNK, NDIM), jnp.float32),
            pltpu.SemaphoreType.DMA,
        ],
        compiler_params=pltpu.CompilerParams(use_tc_tiling_on_sc=False),
    )(_gather_kernel)
    return k(x_flat.reshape(NW, CHUNKS_PER_W, CHUNK), emb_table)


def _mlp_kernel(emb_ref, w1_ref, b1_ref, w2_ref, b2_ref, out_ref, h_ref):
    @pl.when(pl.program_id(0) == 0)
    def _():
        h = jnp.dot(emb_ref[...], w1_ref[...],
                    preferred_element_type=jnp.float32)
        h_ref[...] = jnp.maximum(h + b1_ref[...], 0.0).astype(jnp.bfloat16)

    out_ref[...] = (
        jnp.dot(h_ref[...], w2_ref[...].astype(jnp.bfloat16),
                preferred_element_type=jnp.float32)
        + b2_ref[...]
    )


def _tc_mlp(emb, W1, b1, W2, b2):
    return pl.pallas_call(
        _mlp_kernel,
        grid=(GRID_V,),
        in_specs=[
            pl.BlockSpec((BATCH, CTX * NDIM), lambda i: (0, 0)),
            pl.BlockSpec((CTX * NDIM, HID), lambda i: (0, 0)),
            pl.BlockSpec((1, HID), lambda i: (0, 0)),
            pl.BlockSpec((HID, TILE_V), lambda i: (0, i)),
            pl.BlockSpec((1, TILE_V), lambda i: (0, i)),
        ],
        out_specs=pl.BlockSpec((BATCH, TILE_V), lambda i: (0, i)),
        out_shape=jax.ShapeDtypeStruct((BATCH, VOCAB), jnp.float32),
        scratch_shapes=[pltpu.VMEM((BATCH, HID), jnp.bfloat16)],
    )(emb, W1, b1.reshape(1, HID), W2, b2.reshape(1, VOCAB))


def kernel(x, emb_table, W1, b1, W2, b2):
    rows = _sc_gather(x.reshape(-1), emb_table)
    emb = rows.reshape(BATCH, CTX * NDIM)
    return _tc_mlp(emb, W1, b1, W2, b2)
